# baseline (speedup 1.0000x reference)
import functools

import jax
import jax.numpy as jnp
from jax import lax
from jax.experimental import pallas as pl
from jax.experimental.pallas import tpu as pltpu
from jax.experimental.pallas import tpu_sc as plsc

NC = 2
NS = 16
NW = NC * NS

K = 8
NB = 2


@functools.partial(jax.jit, static_argnums=())
def _gather_rows(ids, table):
    B, = ids.shape
    V, D = table.shape
    b_per_w = B // NW
    nchunk = b_per_w // K

    mesh = plsc.VectorSubcoreMesh(core_axis_name="c", subcore_axis_name="s")

    @functools.partial(
        pl.kernel,
        out_type=jax.ShapeDtypeStruct((B, D), jnp.float32),
        mesh=mesh,
        scratch_types=[
            pltpu.VMEM((b_per_w,), jnp.int32),
            pltpu.VMEM((NB, K, D), jnp.float32),
            pltpu.SemaphoreType.DMA,
            pltpu.SemaphoreType.DMA,
            pltpu.SemaphoreType.DMA,
            pltpu.SemaphoreType.DMA,
        ],
    )
    def body(ids_hbm, table_hbm, out_hbm, idx_v, bufs, g0, g1, w0, w1):
        gsems = (g0, g1)
        wsems = (w0, w1)
        wid = lax.axis_index("s") * NC + lax.axis_index("c")
        base = wid * b_per_w

        pltpu.sync_copy(ids_hbm.at[pl.ds(base, b_per_w)], idx_v)

        def start_gather(c, b):
            pltpu.async_copy(
                table_hbm.at[idx_v.at[pl.ds(c * K, K)]], bufs.at[b], gsems[b]
            )

        def wait_gather(c, b):
            pltpu.make_async_copy(
                table_hbm.at[idx_v.at[pl.ds(c * K, K)]], bufs.at[b], gsems[b]
            ).wait()

        def start_write(c, b):
            pltpu.async_copy(
                bufs.at[b], out_hbm.at[pl.ds(base + c * K, K)], wsems[b]
            )

        def wait_write(c, b):
            pltpu.make_async_copy(
                bufs.at[b], out_hbm.at[pl.ds(base + c * K, K)], wsems[b]
            ).wait()

        start_gather(0, 0)

        @pl.loop(0, nchunk, step=NB)
        def _(c0):
            for b in range(NB):
                c = c0 + b
                nb = (b + 1) % NB
                @pl.when(c + 1 < nchunk)
                def _():
                    @pl.when(c + 1 - NB >= 0)
                    def _():
                        wait_write(c + 1 - NB, nb)

                    start_gather(c + 1, nb)

                wait_gather(c, b)
                start_write(c, b)

        for b in range(NB):
            wait_write(nchunk - NB + b, (nchunk - NB + b) % NB)

    return body(ids, table)


TC_NSEM = 8
TC_DEPTH = 16


@jax.jit
def _gather_rows_tc(ids, table):
    B, = ids.shape
    V, D = table.shape
    ring = TC_NSEM * TC_DEPTH

    def body(ids_ref, table_ref, out_ref, *sems):
        def issue(r, j):
            idx = ids_ref[r]
            pltpu.async_copy(
                table_ref.at[pl.ds(idx, 1)], out_ref.at[pl.ds(r, 1)], sems[j]
            )

        def wait_one(r, j):
            pltpu.make_async_copy(
                table_ref.at[pl.ds(0, 1)], out_ref.at[pl.ds(r, 1)], sems[j]
            ).wait()

        @pl.loop(0, B, step=TC_NSEM)
        def _(r0):
            for j in range(TC_NSEM):
                issue(r0 + j, j)

            @pl.when(r0 >= ring)
            def _():
                for j in range(TC_NSEM):
                    wait_one(r0 - ring + j, j)

        @pl.loop(max(B - ring, 0), B, step=TC_NSEM)
        def _(r0):
            for j in range(TC_NSEM):
                wait_one(r0 + j, j)

    grid_spec = pltpu.PrefetchScalarGridSpec(
        num_scalar_prefetch=1,
        grid=(1,),
        in_specs=[pl.BlockSpec(memory_space=pl.ANY)],
        out_specs=pl.BlockSpec(memory_space=pl.ANY),
        scratch_shapes=[pltpu.SemaphoreType.DMA] * TC_NSEM,
    )
    return pl.pallas_call(
        body,
        grid_spec=grid_spec,
        out_shape=jax.ShapeDtypeStruct((B, D), jnp.float32),
    )(ids, table)


@jax.jit
def _gather_only_probe(ids, table):
    B, = ids.shape
    V, D = table.shape
    b_per_w = B // NW
    nchunk = b_per_w // K

    mesh = plsc.VectorSubcoreMesh(core_axis_name="c", subcore_axis_name="s")

    @functools.partial(
        pl.kernel,
        out_type=jax.ShapeDtypeStruct((B, D), jnp.float32),
        mesh=mesh,
        scratch_types=[
            pltpu.VMEM((b_per_w,), jnp.int32),
            pltpu.VMEM((NB, K, D), jnp.float32),
            pltpu.SemaphoreType.DMA,
            pltpu.SemaphoreType.DMA,
        ],
    )
    def body(ids_hbm, table_hbm, out_hbm, idx_v, bufs, g0, g1):
        gsems = (g0, g1)
        wid = lax.axis_index("s") * NC + lax.axis_index("c")
        base = wid * b_per_w
        pltpu.sync_copy(ids_hbm.at[pl.ds(base, b_per_w)], idx_v)

        def start_gather(c, b):
            pltpu.async_copy(
                table_hbm.at[idx_v.at[pl.ds(c * K, K)]], bufs.at[b], gsems[b]
            )

        def wait_gather(c, b):
            pltpu.make_async_copy(
                table_hbm.at[idx_v.at[pl.ds(c * K, K)]], bufs.at[b], gsems[b]
            ).wait()

        start_gather(0, 0)
        start_gather(1, 1)

        @pl.loop(0, nchunk, step=NB)
        def _(c0):
            for b in range(NB):
                c = c0 + b
                wait_gather(c, b)

                @pl.when(c + NB < nchunk)
                def _():
                    start_gather(c + NB, b)

        pltpu.sync_copy(bufs.at[0], out_hbm.at[pl.ds(base, K)])

    return body(ids, table)


@jax.jit
def _write_only_probe(ids, table):
    B, = ids.shape
    V, D = table.shape
    b_per_w = B // NW
    nchunk = b_per_w // K

    mesh = plsc.VectorSubcoreMesh(core_axis_name="c", subcore_axis_name="s")

    @functools.partial(
        pl.kernel,
        out_type=jax.ShapeDtypeStruct((B, D), jnp.float32),
        mesh=mesh,
        scratch_types=[
            pltpu.VMEM((NB, K, D), jnp.float32),
            pltpu.SemaphoreType.DMA,
            pltpu.SemaphoreType.DMA,
        ],
    )
    def body(ids_hbm, table_hbm, out_hbm, bufs, w0, w1):
        wsems = (w0, w1)
        wid = lax.axis_index("s") * NC + lax.axis_index("c")
        base = wid * b_per_w

        def start_write(c, b):
            pltpu.async_copy(
                bufs.at[b], out_hbm.at[pl.ds(base + c * K, K)], wsems[b]
            )

        def wait_write(c, b):
            pltpu.make_async_copy(
                bufs.at[b], out_hbm.at[pl.ds(base + c * K, K)], wsems[b]
            ).wait()

        start_write(0, 0)
        start_write(1, 1)

        @pl.loop(0, nchunk, step=NB)
        def _(c0):
            for b in range(NB):
                c = c0 + b
                wait_write(c, b)

                @pl.when(c + NB < nchunk)
                def _():
                    start_write(c + NB, b)

    return body(ids, table)


NB3 = 3


@jax.jit
def _gather_rows3(ids, table):
    B, = ids.shape
    V, D = table.shape
    b_per_w = B // NW
    nchunk = b_per_w // K

    mesh = plsc.VectorSubcoreMesh(core_axis_name="c", subcore_axis_name="s")

    @functools.partial(
        pl.kernel,
        out_type=jax.ShapeDtypeStruct((B, D), jnp.float32),
        mesh=mesh,
        scratch_types=[
            pltpu.VMEM((b_per_w,), jnp.int32),
            pltpu.VMEM((NB3, K, D), jnp.float32),
            pltpu.SemaphoreType.DMA,
            pltpu.SemaphoreType.DMA,
            pltpu.SemaphoreType.DMA,
            pltpu.SemaphoreType.DMA,
            pltpu.SemaphoreType.DMA,
            pltpu.SemaphoreType.DMA,
        ],
    )
    def body(ids_hbm, table_hbm, out_hbm, idx_v, bufs, g0, g1, g2, w0, w1, w2):
        gsems = (g0, g1, g2)
        wsems = (w0, w1, w2)
        wid = lax.axis_index("s") * NC + lax.axis_index("c")
        base = wid * b_per_w

        pltpu.sync_copy(ids_hbm.at[pl.ds(base, b_per_w)], idx_v)

        def start_gather(c, b):
            pltpu.async_copy(
                table_hbm.at[idx_v.at[pl.ds(c * K, K)]], bufs.at[b], gsems[b]
            )

        def wait_gather(c, b):
            pltpu.make_async_copy(
                table_hbm.at[idx_v.at[pl.ds(c * K, K)]], bufs.at[b], gsems[b]
            ).wait()

        def start_write(c, b):
            pltpu.async_copy(
                bufs.at[b], out_hbm.at[pl.ds(base + c * K, K)], wsems[b]
            )

        def wait_write(c, b):
            pltpu.make_async_copy(
                bufs.at[b], out_hbm.at[pl.ds(base + c * K, K)], wsems[b]
            ).wait()

        def step(c, b, first=False, last_gather=False):
            b2 = (b + 2) % NB3
            if not first:
                wait_write(c - 1, b2)
            if not last_gather:
                start_gather(c + 2, b2)
            wait_gather(c, b)
            start_write(c, b)

        start_gather(0, 0)
        start_gather(1, 1)
        step(0, 0, first=True)

        @pl.loop(1, nchunk - 7, step=NB3)
        def _(c0):
            for j in range(NB3):
                step(c0 + j, (1 + j) % NB3)

        for c in range(nchunk - 7, nchunk):
            step(c, c % NB3, last_gather=(c + 2 >= nchunk))
        wait_write(nchunk - 1, (nchunk - 1) % NB3)

    return body(ids, table)


def kernel(input_ids, table):
    ids = input_ids.reshape(-1).astype(jnp.int32)
    out = _gather_rows_tc(ids, table)
    return out.reshape(input_ids.shape + (table.shape[1],))


# speedup vs baseline: 35.3225x; 35.3225x over previous
import functools

import jax
import jax.numpy as jnp
from jax import lax
from jax.experimental import pallas as pl
from jax.experimental.pallas import tpu as pltpu
from jax.experimental.pallas import tpu_sc as plsc

NC = 2
NS = 16
NW = NC * NS

K = 8
NB = 2


@functools.partial(jax.jit, static_argnums=())
def _gather_rows(ids, table):
    B, = ids.shape
    V, D = table.shape
    b_per_w = B // NW
    nchunk = b_per_w // K

    mesh = plsc.VectorSubcoreMesh(core_axis_name="c", subcore_axis_name="s")

    @functools.partial(
        pl.kernel,
        out_type=jax.ShapeDtypeStruct((B, D), jnp.float32),
        mesh=mesh,
        scratch_types=[
            pltpu.VMEM((b_per_w,), jnp.int32),
            pltpu.VMEM((NB, K, D), jnp.float32),
            pltpu.SemaphoreType.DMA,
            pltpu.SemaphoreType.DMA,
            pltpu.SemaphoreType.DMA,
            pltpu.SemaphoreType.DMA,
        ],
    )
    def body(ids_hbm, table_hbm, out_hbm, idx_v, bufs, g0, g1, w0, w1):
        gsems = (g0, g1)
        wsems = (w0, w1)
        wid = lax.axis_index("s") * NC + lax.axis_index("c")
        base = wid * b_per_w

        pltpu.sync_copy(ids_hbm.at[pl.ds(base, b_per_w)], idx_v)

        def start_gather(c, b):
            pltpu.async_copy(
                table_hbm.at[idx_v.at[pl.ds(c * K, K)]], bufs.at[b], gsems[b]
            )

        def wait_gather(c, b):
            pltpu.make_async_copy(
                table_hbm.at[idx_v.at[pl.ds(c * K, K)]], bufs.at[b], gsems[b]
            ).wait()

        def start_write(c, b):
            pltpu.async_copy(
                bufs.at[b], out_hbm.at[pl.ds(base + c * K, K)], wsems[b]
            )

        def wait_write(c, b):
            pltpu.make_async_copy(
                bufs.at[b], out_hbm.at[pl.ds(base + c * K, K)], wsems[b]
            ).wait()

        start_gather(0, 0)

        @pl.loop(0, nchunk, step=NB)
        def _(c0):
            for b in range(NB):
                c = c0 + b
                nb = (b + 1) % NB
                @pl.when(c + 1 < nchunk)
                def _():
                    @pl.when(c + 1 - NB >= 0)
                    def _():
                        wait_write(c + 1 - NB, nb)

                    start_gather(c + 1, nb)

                wait_gather(c, b)
                start_write(c, b)

        for b in range(NB):
            wait_write(nchunk - NB + b, (nchunk - NB + b) % NB)

    return body(ids, table)


TC_NSEM = 8
TC_DEPTH = 16


@jax.jit
def _gather_rows_tc(ids, table):
    B, = ids.shape
    V, D = table.shape
    ring = TC_NSEM * TC_DEPTH

    def body(ids_ref, table_ref, out_ref, *sems):
        def issue(r, j):
            idx = ids_ref[r]
            pltpu.async_copy(
                table_ref.at[pl.ds(idx, 1)], out_ref.at[pl.ds(r, 1)], sems[j]
            )

        def wait_one(r, j):
            pltpu.make_async_copy(
                table_ref.at[pl.ds(0, 1)], out_ref.at[pl.ds(r, 1)], sems[j]
            ).wait()

        @pl.loop(0, B, step=TC_NSEM)
        def _(r0):
            for j in range(TC_NSEM):
                issue(r0 + j, j)

            @pl.when(r0 >= ring)
            def _():
                for j in range(TC_NSEM):
                    wait_one(r0 - ring + j, j)

        @pl.loop(max(B - ring, 0), B, step=TC_NSEM)
        def _(r0):
            for j in range(TC_NSEM):
                wait_one(r0 + j, j)

    grid_spec = pltpu.PrefetchScalarGridSpec(
        num_scalar_prefetch=1,
        grid=(1,),
        in_specs=[pl.BlockSpec(memory_space=pl.ANY)],
        out_specs=pl.BlockSpec(memory_space=pl.ANY),
        scratch_shapes=[pltpu.SemaphoreType.DMA] * TC_NSEM,
    )
    return pl.pallas_call(
        body,
        grid_spec=grid_spec,
        out_shape=jax.ShapeDtypeStruct((B, D), jnp.float32),
    )(ids, table)


@jax.jit
def _gather_only_probe(ids, table):
    B, = ids.shape
    V, D = table.shape
    b_per_w = B // NW
    nchunk = b_per_w // K

    mesh = plsc.VectorSubcoreMesh(core_axis_name="c", subcore_axis_name="s")

    @functools.partial(
        pl.kernel,
        out_type=jax.ShapeDtypeStruct((B, D), jnp.float32),
        mesh=mesh,
        scratch_types=[
            pltpu.VMEM((b_per_w,), jnp.int32),
            pltpu.VMEM((NB, K, D), jnp.float32),
            pltpu.SemaphoreType.DMA,
            pltpu.SemaphoreType.DMA,
        ],
    )
    def body(ids_hbm, table_hbm, out_hbm, idx_v, bufs, g0, g1):
        gsems = (g0, g1)
        wid = lax.axis_index("s") * NC + lax.axis_index("c")
        base = wid * b_per_w
        pltpu.sync_copy(ids_hbm.at[pl.ds(base, b_per_w)], idx_v)

        def start_gather(c, b):
            pltpu.async_copy(
                table_hbm.at[idx_v.at[pl.ds(c * K, K)]], bufs.at[b], gsems[b]
            )

        def wait_gather(c, b):
            pltpu.make_async_copy(
                table_hbm.at[idx_v.at[pl.ds(c * K, K)]], bufs.at[b], gsems[b]
            ).wait()

        start_gather(0, 0)
        start_gather(1, 1)

        @pl.loop(0, nchunk, step=NB)
        def _(c0):
            for b in range(NB):
                c = c0 + b
                wait_gather(c, b)

                @pl.when(c + NB < nchunk)
                def _():
                    start_gather(c + NB, b)

        pltpu.sync_copy(bufs.at[0], out_hbm.at[pl.ds(base, K)])

    return body(ids, table)


@jax.jit
def _write_only_probe(ids, table):
    B, = ids.shape
    V, D = table.shape
    b_per_w = B // NW
    nchunk = b_per_w // K

    mesh = plsc.VectorSubcoreMesh(core_axis_name="c", subcore_axis_name="s")

    @functools.partial(
        pl.kernel,
        out_type=jax.ShapeDtypeStruct((B, D), jnp.float32),
        mesh=mesh,
        scratch_types=[
            pltpu.VMEM((NB, K, D), jnp.float32),
            pltpu.SemaphoreType.DMA,
            pltpu.SemaphoreType.DMA,
        ],
    )
    def body(ids_hbm, table_hbm, out_hbm, bufs, w0, w1):
        wsems = (w0, w1)
        wid = lax.axis_index("s") * NC + lax.axis_index("c")
        base = wid * b_per_w

        def start_write(c, b):
            pltpu.async_copy(
                bufs.at[b], out_hbm.at[pl.ds(base + c * K, K)], wsems[b]
            )

        def wait_write(c, b):
            pltpu.make_async_copy(
                bufs.at[b], out_hbm.at[pl.ds(base + c * K, K)], wsems[b]
            ).wait()

        start_write(0, 0)
        start_write(1, 1)

        @pl.loop(0, nchunk, step=NB)
        def _(c0):
            for b in range(NB):
                c = c0 + b
                wait_write(c, b)

                @pl.when(c + NB < nchunk)
                def _():
                    start_write(c + NB, b)

    return body(ids, table)


NB3 = 3


@jax.jit
def _gather_rows3(ids, table):
    B, = ids.shape
    V, D = table.shape
    b_per_w = B // NW
    nchunk = b_per_w // K

    mesh = plsc.VectorSubcoreMesh(core_axis_name="c", subcore_axis_name="s")

    @functools.partial(
        pl.kernel,
        out_type=jax.ShapeDtypeStruct((B, D), jnp.float32),
        mesh=mesh,
        scratch_types=[
            pltpu.VMEM((b_per_w,), jnp.int32),
            pltpu.VMEM((NB3, K, D), jnp.float32),
            pltpu.SemaphoreType.DMA,
            pltpu.SemaphoreType.DMA,
            pltpu.SemaphoreType.DMA,
            pltpu.SemaphoreType.DMA,
            pltpu.SemaphoreType.DMA,
            pltpu.SemaphoreType.DMA,
        ],
    )
    def body(ids_hbm, table_hbm, out_hbm, idx_v, bufs, g0, g1, g2, w0, w1, w2):
        gsems = (g0, g1, g2)
        wsems = (w0, w1, w2)
        wid = lax.axis_index("s") * NC + lax.axis_index("c")
        base = wid * b_per_w

        pltpu.sync_copy(ids_hbm.at[pl.ds(base, b_per_w)], idx_v)

        def start_gather(c, b):
            pltpu.async_copy(
                table_hbm.at[idx_v.at[pl.ds(c * K, K)]], bufs.at[b], gsems[b]
            )

        def wait_gather(c, b):
            pltpu.make_async_copy(
                table_hbm.at[idx_v.at[pl.ds(c * K, K)]], bufs.at[b], gsems[b]
            ).wait()

        def start_write(c, b):
            pltpu.async_copy(
                bufs.at[b], out_hbm.at[pl.ds(base + c * K, K)], wsems[b]
            )

        def wait_write(c, b):
            pltpu.make_async_copy(
                bufs.at[b], out_hbm.at[pl.ds(base + c * K, K)], wsems[b]
            ).wait()

        def step(c, b, first=False, last_gather=False):
            b2 = (b + 2) % NB3
            if not first:
                wait_write(c - 1, b2)
            if not last_gather:
                start_gather(c + 2, b2)
            wait_gather(c, b)
            start_write(c, b)

        start_gather(0, 0)
        start_gather(1, 1)
        step(0, 0, first=True)

        @pl.loop(1, nchunk - 7, step=NB3)
        def _(c0):
            for j in range(NB3):
                step(c0 + j, (1 + j) % NB3)

        for c in range(nchunk - 7, nchunk):
            step(c, c % NB3, last_gather=(c + 2 >= nchunk))
        wait_write(nchunk - 1, (nchunk - 1) % NB3)

    return body(ids, table)


@jax.jit
def _gather_rows_spmem(ids, table):
    B, = ids.shape
    V, D = table.shape
    b_per_w = B // NW
    nchunk = b_per_w // K

    mesh = plsc.VectorSubcoreMesh(core_axis_name="c", subcore_axis_name="s")

    H = 2
    KH = K // H

    @functools.partial(
        pl.kernel,
        out_type=jax.ShapeDtypeStruct((B, D), jnp.float32),
        mesh=mesh,
        scratch_types=[
            pltpu.VMEM((b_per_w,), jnp.int32),
            pltpu.VMEM((NB, K, D), jnp.float32),
            pltpu.VMEM_SHARED((NS, H, KH, D), jnp.float32),
            pltpu.SemaphoreType.DMA,
            pltpu.SemaphoreType.DMA,
            pltpu.SemaphoreType.DMA,
            pltpu.SemaphoreType.DMA,
            pltpu.SemaphoreType.DMA,
            pltpu.SemaphoreType.DMA,
        ],
    )
    def body(ids_hbm, table_hbm, out_hbm, idx_v, bufs, shared,
             g0, g1, x0, x1, w0, w1):
        gsems = (g0, g1)
        xsems = (x0, x1)
        wsems = (w0, w1)
        sid = lax.axis_index("s")
        wid = sid * NC + lax.axis_index("c")
        base = wid * b_per_w

        pltpu.sync_copy(ids_hbm.at[pl.ds(base, b_per_w)], idx_v)

        def start_gather(c, b):
            pltpu.async_copy(
                table_hbm.at[idx_v.at[pl.ds(c * K, K)]], bufs.at[b], gsems[b]
            )

        def wait_gather(c, b):
            pltpu.make_async_copy(
                table_hbm.at[idx_v.at[pl.ds(c * K, K)]], bufs.at[b], gsems[b]
            ).wait()

        def start_xbar(c, b, h):
            pltpu.async_copy(
                bufs.at[b].at[pl.ds(h * KH, KH)], shared.at[sid].at[h],
                xsems[h],
            )

        def wait_xbar(c, b, h):
            pltpu.make_async_copy(
                bufs.at[b].at[pl.ds(h * KH, KH)], shared.at[sid].at[h],
                xsems[h],
            ).wait()

        def start_write(c, h):
            pltpu.async_copy(
                shared.at[sid].at[h],
                out_hbm.at[pl.ds(base + c * K + h * KH, KH)], wsems[h],
            )

        def wait_write(c, h):
            pltpu.make_async_copy(
                shared.at[sid].at[h],
                out_hbm.at[pl.ds(base + c * K + h * KH, KH)], wsems[h],
            ).wait()

        start_gather(0, 0)

        @pl.loop(0, nchunk, step=NB)
        def _(c0):
            for b in range(NB):
                c = c0 + b
                nb = (b + 1) % NB

                @pl.when(c - 1 >= 0)
                def _():
                    for h in range(H):
                        wait_xbar(c - 1, nb, h)
                        start_write(c - 1, h)

                @pl.when(c + 1 < nchunk)
                def _():
                    start_gather(c + 1, nb)

                wait_gather(c, b)
                for h in range(H):
                    @pl.when(c - 1 >= 0)
                    def _():
                        wait_write(c - 1, h)

                    start_xbar(c, b, h)

        last = nchunk - 1
        for h in range(H):
            wait_xbar(last, last % NB, h)
            start_write(last, h)
            wait_write(last, h)

    return body(ids, table)


def kernel(input_ids, table):
    ids = input_ids.reshape(-1).astype(jnp.int32)
    out = _gather_rows_spmem(ids, table)
    return out.reshape(input_ids.shape + (table.shape[1],))


# speedup vs baseline: 35.9059x; 1.0165x over previous
import functools

import jax
import jax.numpy as jnp
from jax import lax
from jax.experimental import pallas as pl
from jax.experimental.pallas import tpu as pltpu
from jax.experimental.pallas import tpu_sc as plsc

NC = 2
NS = 16
NW = NC * NS

K = 8
NB = 2


@jax.jit
def _gather_rows(ids, table):
    B, = ids.shape
    V, D = table.shape
    b_per_w = B // NW
    nchunk = b_per_w // K

    mesh = plsc.VectorSubcoreMesh(core_axis_name="c", subcore_axis_name="s")

    @functools.partial(
        pl.kernel,
        out_type=jax.ShapeDtypeStruct((B, D), jnp.float32),
        mesh=mesh,
        scratch_types=[
            pltpu.VMEM((b_per_w,), jnp.int32),
            pltpu.VMEM((NB, K, D), jnp.float32),
            pltpu.SemaphoreType.DMA,
            pltpu.SemaphoreType.DMA,
            pltpu.SemaphoreType.DMA,
            pltpu.SemaphoreType.DMA,
        ],
    )
    def body(ids_hbm, table_hbm, out_hbm, idx_v, bufs, g0, g1, w0, w1):
        gsems = (g0, g1)
        wsems = (w0, w1)
        wid = lax.axis_index("s") * NC + lax.axis_index("c")
        base = wid * b_per_w

        pltpu.sync_copy(ids_hbm.at[pl.ds(base, b_per_w)], idx_v)

        def start_gather(c, b):
            pltpu.async_copy(
                table_hbm.at[idx_v.at[pl.ds(c * K, K)]], bufs.at[b], gsems[b]
            )

        def wait_gather(c, b):
            pltpu.make_async_copy(
                table_hbm.at[idx_v.at[pl.ds(c * K, K)]], bufs.at[b], gsems[b]
            ).wait()

        def start_write(c, b):
            pltpu.async_copy(
                bufs.at[b], out_hbm.at[pl.ds(base + c * K, K)], wsems[b]
            )

        def wait_write(c, b):
            pltpu.make_async_copy(
                bufs.at[b], out_hbm.at[pl.ds(base + c * K, K)], wsems[b]
            ).wait()

        start_gather(0, 0)

        @pl.loop(0, nchunk, step=NB)
        def _(c0):
            for b in range(NB):
                c = c0 + b
                nb = (b + 1) % NB
                @pl.when(c + 1 < nchunk)
                def _():
                    @pl.when(c + 1 - NB >= 0)
                    def _():
                        wait_write(c + 1 - NB, nb)

                    start_gather(c + 1, nb)

                wait_gather(c, b)
                start_write(c, b)

        for b in range(NB):
            wait_write(nchunk - NB + b, (nchunk - NB + b) % NB)

    return body(ids, table)


def kernel(input_ids, table):
    ids = input_ids.reshape(-1).astype(jnp.int32)
    out = _gather_rows(ids, table)
    return out.reshape(input_ids.shape + (table.shape[1],))
